# baseline (speedup 1.0000x reference)
import functools

import jax
import jax.numpy as jnp
from jax import lax
from jax.experimental import pallas as pl
from jax.experimental.pallas import tpu as pltpu
from jax.experimental.pallas import tpu_sc as plsc

N_VERT = 10000
N_HE = 320000
D = 128

NC = 2
NS = 16
NW = NC * NS
HE_PER_W = N_HE // NW
CH = 120
NV = HE_PER_W // CH
TAIL = HE_PER_W - NV * CH
TAIL_OFF = NV * CH
NBUF = 3
G = 200
V_CHUNKS = N_VERT // G
RR = (V_CHUNKS + NS - 1) // NS


def _sc_partial_sums(x, idx4, idx_t, zrows):
    mesh = plsc.VectorSubcoreMesh(core_axis_name="c", subcore_axis_name="s")

    @functools.partial(
        pl.kernel,
        out_type=jax.ShapeDtypeStruct((NC, N_VERT, D), jnp.float32),
        mesh=mesh,
        scratch_types=[
            [pltpu.VMEM((1, CH), jnp.int32) for _ in range(NBUF)],
            pltpu.VMEM((1, TAIL), jnp.int32),
            [pltpu.VMEM((CH, D), jnp.float32) for _ in range(NBUF)],
            pltpu.VMEM_SHARED((N_VERT, D), jnp.float32),
            [pltpu.SemaphoreType.DMA for _ in range(NBUF)],
            [pltpu.SemaphoreType.DMA for _ in range(NBUF)],
        ],
    )
    def k(x_hbm, idx_hbm, idxt_hbm, z_hbm, out_hbm,
          icb, itb, bufs, acc, gsems, isems):
        c = lax.axis_index("c")
        s = lax.axis_index("s")
        wid = c * NS + s
        he_base = wid * HE_PER_W

        def gather(v, b):
            pltpu.async_copy(
                x_hbm.at[pl.ds(he_base + v * CH, CH)], bufs[b], gsems[b])
            pltpu.async_copy(idx_hbm.at[wid, v], icb[b], isems[b])

        def wait_gather(b):
            pltpu.make_async_copy(
                x_hbm.at[pl.ds(0, CH)], bufs[b], gsems[b]).wait()
            pltpu.make_async_copy(idx_hbm.at[wid, 0], icb[b], isems[b]).wait()

        def scatter(b):
            pltpu.sync_copy(bufs[b], acc.at[icb[b].at[0]], add=True)

        pltpu.sync_copy(idxt_hbm.at[wid], itb)
        gather(1, 1)
        gather(2, 2)

        for j in range(RR):
            cid = s + j * NS

            @pl.when(cid < V_CHUNKS)
            def _():
                pltpu.sync_copy(z_hbm, acc.at[pl.ds(cid * G, G)])
        plsc.subcore_barrier()

        gather(0, 0)

        def visit(v, b):
            @pl.when(v < NV)
            def _():
                wait_gather(b)
                scatter(b)

                @pl.when(v + NBUF < NV)
                def _():
                    gather(v + NBUF, b)

        def tri_body(i, _):
            for b in range(NBUF):
                visit(NBUF * i + b, b)
            return 0

        lax.fori_loop(0, (NV + NBUF - 1) // NBUF, tri_body, 0)
        pltpu.sync_copy(x_hbm.at[pl.ds(he_base + TAIL_OFF, TAIL)],
                        bufs[0].at[pl.ds(0, TAIL)])
        pltpu.sync_copy(bufs[0].at[pl.ds(0, TAIL)],
                        acc.at[itb.at[0]], add=True)
        plsc.subcore_barrier()

        for j in range(RR):
            cid = s + j * NS

            @pl.when(cid < V_CHUNKS)
            def _():
                pltpu.sync_copy(acc.at[pl.ds(cid * G, G)],
                                out_hbm.at[c, pl.ds(cid * G, G)])

    return k(x, idx4, idx_t, zrows)


def _combine_body(p_ref, v_ref, o_ref):
    o_ref[...] = (p_ref[0] + p_ref[1]) / v_ref[...]


def _combine(partials, valence):
    rb = 1000
    grid = N_VERT // rb
    return pl.pallas_call(
        _combine_body,
        grid=(grid,),
        in_specs=[
            pl.BlockSpec((NC, rb, D), lambda i: (0, i, 0)),
            pl.BlockSpec((rb, 1), lambda i: (i, 0)),
        ],
        out_specs=pl.BlockSpec((rb, D), lambda i: (i, 0)),
        out_shape=jax.ShapeDtypeStruct((N_VERT, D), jnp.float32),
    )(partials, valence.reshape(N_VERT, 1))


def kernel(x, half_edge_src, vertex_valence):
    idx_all = half_edge_src.astype(jnp.int32).reshape(NW, HE_PER_W)
    idx4 = idx_all[:, :TAIL_OFF].reshape(NW, NV, 1, CH)
    idx_t = idx_all[:, TAIL_OFF:].reshape(NW, 1, TAIL)
    zrows = jnp.zeros((G, D), jnp.float32)
    partials = _sc_partial_sums(x, idx4, idx_t, zrows)
    return _combine(partials, vertex_valence)


# speedup vs baseline: 1.0461x; 1.0461x over previous
import functools

import jax
import jax.numpy as jnp
from jax import lax
from jax.experimental import pallas as pl
from jax.experimental.pallas import tpu as pltpu
from jax.experimental.pallas import tpu_sc as plsc

N_VERT = 10000
N_HE = 320000
D = 128

NC = 2
NS = 16
NW = NC * NS
HE_PER_W = N_HE // NW
CH = 120
NV = HE_PER_W // CH
TAIL = HE_PER_W - NV * CH
TAIL_OFF = NV * CH
NBUF = 3
G = 80
V_CHUNKS = N_VERT // G
RR = (V_CHUNKS + NS - 1) // NS


def _sc_partial_sums(x, idx4, idx_t, zrows):
    mesh = plsc.VectorSubcoreMesh(core_axis_name="c", subcore_axis_name="s")

    @functools.partial(
        pl.kernel,
        out_type=jax.ShapeDtypeStruct((NC, N_VERT, D), jnp.float32),
        mesh=mesh,
        scratch_types=[
            [pltpu.VMEM((1, CH), jnp.int32) for _ in range(NBUF)],
            pltpu.VMEM((1, TAIL), jnp.int32),
            [pltpu.VMEM((CH, D), jnp.float32) for _ in range(NBUF)],
            pltpu.VMEM_SHARED((N_VERT, D), jnp.float32),
            [pltpu.SemaphoreType.DMA for _ in range(NBUF)],
            [pltpu.SemaphoreType.DMA for _ in range(NBUF)],
        ],
    )
    def k(x_hbm, idx_hbm, idxt_hbm, z_hbm, out_hbm,
          icb, itb, bufs, acc, gsems, isems):
        c = lax.axis_index("c")
        s = lax.axis_index("s")
        wid = c * NS + s
        he_base = wid * HE_PER_W

        def gather(v, b):
            pltpu.async_copy(
                x_hbm.at[pl.ds(he_base + v * CH, CH)], bufs[b], gsems[b])
            pltpu.async_copy(idx_hbm.at[wid, v], icb[b], isems[b])

        def wait_gather(b):
            pltpu.make_async_copy(
                x_hbm.at[pl.ds(0, CH)], bufs[b], gsems[b]).wait()
            pltpu.make_async_copy(idx_hbm.at[wid, 0], icb[b], isems[b]).wait()

        def scatter(b):
            pltpu.sync_copy(bufs[b], acc.at[icb[b].at[0]], add=True)

        pltpu.sync_copy(idxt_hbm.at[wid], itb)
        pltpu.sync_copy(z_hbm, bufs[0].at[pl.ds(0, G)])
        gather(1, 1)
        gather(2, 2)

        for j in range(RR):
            cid = s + j * NS

            @pl.when(cid < V_CHUNKS)
            def _():
                pltpu.sync_copy(bufs[0].at[pl.ds(0, G)],
                                acc.at[pl.ds(cid * G, G)])
        plsc.subcore_barrier()

        gather(0, 0)

        def visit(v, b):
            @pl.when(v < NV)
            def _():
                wait_gather(b)
                scatter(b)

                @pl.when(v + NBUF < NV)
                def _():
                    gather(v + NBUF, b)

        def tri_body(i, _):
            for b in range(NBUF):
                visit(NBUF * i + b, b)
            return 0

        lax.fori_loop(0, (NV + NBUF - 1) // NBUF, tri_body, 0)
        pltpu.sync_copy(x_hbm.at[pl.ds(he_base + TAIL_OFF, TAIL)],
                        bufs[0].at[pl.ds(0, TAIL)])
        pltpu.sync_copy(bufs[0].at[pl.ds(0, TAIL)],
                        acc.at[itb.at[0]], add=True)
        plsc.subcore_barrier()

        for j in range(RR):
            cid = s + j * NS

            @pl.when(cid < V_CHUNKS)
            def _():
                pltpu.sync_copy(acc.at[pl.ds(cid * G, G)],
                                out_hbm.at[c, pl.ds(cid * G, G)])

    return k(x, idx4, idx_t, zrows)


def _combine_body(p_ref, v_ref, o_ref):
    o_ref[...] = (p_ref[0] + p_ref[1]) / v_ref[...]


def _combine(partials, valence):
    rb = 1000
    grid = N_VERT // rb
    return pl.pallas_call(
        _combine_body,
        grid=(grid,),
        in_specs=[
            pl.BlockSpec((NC, rb, D), lambda i: (0, i, 0)),
            pl.BlockSpec((rb, 1), lambda i: (i, 0)),
        ],
        out_specs=pl.BlockSpec((rb, D), lambda i: (i, 0)),
        out_shape=jax.ShapeDtypeStruct((N_VERT, D), jnp.float32),
    )(partials, valence.reshape(N_VERT, 1))


def kernel(x, half_edge_src, vertex_valence):
    idx_all = half_edge_src.astype(jnp.int32).reshape(NW, HE_PER_W)
    idx4 = idx_all[:, :TAIL_OFF].reshape(NW, NV, 1, CH)
    idx_t = idx_all[:, TAIL_OFF:].reshape(NW, 1, TAIL)
    zrows = jnp.zeros((G, D), jnp.float32)
    partials = _sc_partial_sums(x, idx4, idx_t, zrows)
    return _combine(partials, vertex_valence)


# speedup vs baseline: 1.0650x; 1.0181x over previous
import functools

import jax
import jax.numpy as jnp
from jax import lax
from jax.experimental import pallas as pl
from jax.experimental.pallas import tpu as pltpu
from jax.experimental.pallas import tpu_sc as plsc

N_VERT = 10000
N_HE = 320000
D = 128

NC = 2
NS = 16
NW = NC * NS
HE_PER_W = N_HE // NW
CH = 120
NV = HE_PER_W // CH
TAIL = HE_PER_W - NV * CH
TAIL_OFF = NV * CH
NBUF = 3
G = 80
V_CHUNKS = N_VERT // G
RR = (V_CHUNKS + NS - 1) // NS


def _sc_partial_sums(x, idx4, idx_t, zrows):
    mesh = plsc.VectorSubcoreMesh(core_axis_name="c", subcore_axis_name="s")

    @functools.partial(
        pl.kernel,
        out_type=jax.ShapeDtypeStruct((NC, N_VERT, D), jnp.float32),
        mesh=mesh,
        scratch_types=[
            [pltpu.VMEM((1, CH), jnp.int32) for _ in range(NBUF)],
            pltpu.VMEM((1, TAIL), jnp.int32),
            [pltpu.VMEM((CH, D), jnp.float32) for _ in range(NBUF)],
            pltpu.VMEM_SHARED((N_VERT, D), jnp.float32),
            [pltpu.SemaphoreType.DMA for _ in range(NBUF)],
            [pltpu.SemaphoreType.DMA for _ in range(NBUF)],
        ],
    )
    def k(x_hbm, idx_hbm, idxt_hbm, z_hbm, out_hbm,
          icb, itb, bufs, acc, gsems, isems):
        c = lax.axis_index("c")
        s = lax.axis_index("s")
        wid = c * NS + s
        he_base = wid * HE_PER_W

        def gather(v, b):
            pltpu.async_copy(
                x_hbm.at[pl.ds(he_base + v * CH, CH)], bufs[b], gsems[b])
            pltpu.async_copy(idx_hbm.at[wid, v], icb[b], isems[b])

        def wait_gather(b):
            pltpu.make_async_copy(
                x_hbm.at[pl.ds(0, CH)], bufs[b], gsems[b]).wait()
            pltpu.make_async_copy(idx_hbm.at[wid, 0], icb[b], isems[b]).wait()

        def scatter(b):
            pltpu.sync_copy(bufs[b], acc.at[icb[b].at[0]], add=True)

        pltpu.sync_copy(idxt_hbm.at[wid], itb)
        pltpu.sync_copy(z_hbm, bufs[0].at[pl.ds(0, G)])
        gather(1, 1)
        gather(2, 2)

        for j in range(RR):
            cid = s + j * NS

            @pl.when(cid < V_CHUNKS)
            def _():
                pltpu.sync_copy(bufs[0].at[pl.ds(0, G)],
                                acc.at[pl.ds(cid * G, G)])
        plsc.subcore_barrier()

        gather(0, 0)

        def visit(v, b):
            @pl.when(v < NV)
            def _():
                wait_gather(b)
                scatter(b)

                @pl.when(v + NBUF < NV)
                def _():
                    gather(v + NBUF, b)

        def tri_body(i, _):
            for b in range(NBUF):
                visit(NBUF * i + b, b)
            return 0

        lax.fori_loop(0, (NV + NBUF - 1) // NBUF, tri_body, 0)
        pltpu.sync_copy(x_hbm.at[pl.ds(he_base + TAIL_OFF, TAIL)],
                        bufs[0].at[pl.ds(0, TAIL)])
        pltpu.sync_copy(bufs[0].at[pl.ds(0, TAIL)],
                        acc.at[itb.at[0]], add=True)
        plsc.subcore_barrier()

        for j in range(RR):
            cid = s + j * NS

            @pl.when(cid < V_CHUNKS)
            def _():
                pltpu.sync_copy(acc.at[pl.ds(cid * G, G)],
                                out_hbm.at[c, pl.ds(cid * G, G)])

    return k(x, idx4, idx_t, zrows)


def _combine_body(p_ref, v_ref, o_ref):
    o_ref[...] = (p_ref[0] + p_ref[1]) / v_ref[...]


def _combine(partials, valence):
    rb = 2000
    grid = N_VERT // rb
    return pl.pallas_call(
        _combine_body,
        grid=(grid,),
        in_specs=[
            pl.BlockSpec((NC, rb, D), lambda i: (0, i, 0)),
            pl.BlockSpec((rb, 1), lambda i: (i, 0)),
        ],
        out_specs=pl.BlockSpec((rb, D), lambda i: (i, 0)),
        out_shape=jax.ShapeDtypeStruct((N_VERT, D), jnp.float32),
    )(partials, valence.reshape(N_VERT, 1))


def kernel(x, half_edge_src, vertex_valence):
    idx_all = half_edge_src.astype(jnp.int32).reshape(NW, HE_PER_W)
    idx4 = idx_all[:, :TAIL_OFF].reshape(NW, NV, 1, CH)
    idx_t = idx_all[:, TAIL_OFF:].reshape(NW, 1, TAIL)
    zrows = jnp.zeros((G, D), jnp.float32)
    partials = _sc_partial_sums(x, idx4, idx_t, zrows)
    return _combine(partials, vertex_valence)


# speedup vs baseline: 1.0746x; 1.0090x over previous
import functools

import jax
import jax.numpy as jnp
from jax import lax
from jax.experimental import pallas as pl
from jax.experimental.pallas import tpu as pltpu
from jax.experimental.pallas import tpu_sc as plsc

N_VERT = 10000
N_HE = 320000
D = 128

NC = 2
NS = 16
NW = NC * NS
HE_PER_W = N_HE // NW
CH = 120
NV = HE_PER_W // CH
TAIL = HE_PER_W - NV * CH
TAIL_OFF = NV * CH
NBUF = 3
G = 80
V_CHUNKS = N_VERT // G
RR = (V_CHUNKS + NS - 1) // NS


def _sc_partial_sums(x, idx4, idx_t, zrows):
    mesh = plsc.VectorSubcoreMesh(core_axis_name="c", subcore_axis_name="s")

    @functools.partial(
        pl.kernel,
        out_type=jax.ShapeDtypeStruct((NC, N_VERT, D), jnp.float32),
        mesh=mesh,
        scratch_types=[
            [pltpu.VMEM((1, CH), jnp.int32) for _ in range(NBUF)],
            pltpu.VMEM((1, TAIL), jnp.int32),
            [pltpu.VMEM((CH, D), jnp.float32) for _ in range(NBUF)],
            pltpu.VMEM_SHARED((N_VERT, D), jnp.float32),
            [pltpu.SemaphoreType.DMA for _ in range(NBUF)],
            [pltpu.SemaphoreType.DMA for _ in range(NBUF)],
        ],
    )
    def k(x_hbm, idx_hbm, idxt_hbm, z_hbm, out_hbm,
          icb, itb, bufs, acc, gsems, isems):
        c = lax.axis_index("c")
        s = lax.axis_index("s")
        wid = c * NS + s
        he_base = wid * HE_PER_W

        def gather(v, b):
            pltpu.async_copy(
                x_hbm.at[pl.ds(he_base + v * CH, CH)], bufs[b], gsems[b])
            pltpu.async_copy(idx_hbm.at[wid, v], icb[b], isems[b])

        def wait_gather(b):
            pltpu.make_async_copy(
                x_hbm.at[pl.ds(0, CH)], bufs[b], gsems[b]).wait()
            pltpu.make_async_copy(idx_hbm.at[wid, 0], icb[b], isems[b]).wait()

        def scatter(b):
            pltpu.sync_copy(bufs[b], acc.at[icb[b].at[0]], add=True)

        pltpu.sync_copy(idxt_hbm.at[wid], itb)
        pltpu.sync_copy(z_hbm, bufs[0].at[pl.ds(0, G)])
        gather(1, 1)
        gather(2, 2)

        for j in range(RR):
            cid = s + j * NS

            @pl.when(cid < V_CHUNKS)
            def _():
                pltpu.sync_copy(bufs[0].at[pl.ds(0, G)],
                                acc.at[pl.ds(cid * G, G)])
        plsc.subcore_barrier()

        gather(0, 0)

        def visit(v, b):
            @pl.when(v < NV)
            def _():
                wait_gather(b)
                scatter(b)

                @pl.when(v + NBUF < NV)
                def _():
                    gather(v + NBUF, b)

        def tri_body(i, _):
            for b in range(NBUF):
                visit(NBUF * i + b, b)
            return 0

        lax.fori_loop(0, (NV + NBUF - 1) // NBUF, tri_body, 0)
        pltpu.sync_copy(x_hbm.at[pl.ds(he_base + TAIL_OFF, TAIL)],
                        bufs[0].at[pl.ds(0, TAIL)])
        pltpu.sync_copy(bufs[0].at[pl.ds(0, TAIL)],
                        acc.at[itb.at[0]], add=True)
        plsc.subcore_barrier()

        for j in range(RR):
            cid = s + j * NS

            @pl.when(cid < V_CHUNKS)
            def _():
                pltpu.sync_copy(acc.at[pl.ds(cid * G, G)],
                                out_hbm.at[c, pl.ds(cid * G, G)])

    return k(x, idx4, idx_t, zrows)


def _combine_body(p_ref, v_ref, o_ref):
    o_ref[...] = (p_ref[0] + p_ref[1]) / v_ref[...]


def _combine(partials, valence):
    rb = 5000
    grid = N_VERT // rb
    return pl.pallas_call(
        _combine_body,
        grid=(grid,),
        in_specs=[
            pl.BlockSpec((NC, rb, D), lambda i: (0, i, 0)),
            pl.BlockSpec((rb, 1), lambda i: (i, 0)),
        ],
        out_specs=pl.BlockSpec((rb, D), lambda i: (i, 0)),
        out_shape=jax.ShapeDtypeStruct((N_VERT, D), jnp.float32),
    )(partials, valence.reshape(N_VERT, 1))


def kernel(x, half_edge_src, vertex_valence):
    idx_all = half_edge_src.astype(jnp.int32).reshape(NW, HE_PER_W)
    idx4 = idx_all[:, :TAIL_OFF].reshape(NW, NV, 1, CH)
    idx_t = idx_all[:, TAIL_OFF:].reshape(NW, 1, TAIL)
    zrows = jnp.zeros((G, D), jnp.float32)
    partials = _sc_partial_sums(x, idx4, idx_t, zrows)
    return _combine(partials, vertex_valence)
